# baseline (speedup 1.0000x reference)
import functools

import jax
import jax.numpy as jnp
from jax import lax
from jax.experimental import pallas as pl
from jax.experimental.pallas import tpu as pltpu
from jax.experimental.pallas import tpu_sc as plsc

def _lane_gather(x, idx):
    dn = lax.GatherDimensionNumbers(
        offset_dims=(), collapsed_slice_dims=(0,), start_index_map=(0,))
    return lax.gather(x, idx[:, None], dn, slice_sizes=(1,),
                      mode=lax.GatherScatterMode.PROMISE_IN_BOUNDS)


N = 10000
E = 320000
D = 128
L = 10

NC = 2
NS = 16
NW = NC * NS
EPB = E // NW
EPBP = 10240
NSTG = 10
S0 = 5008
S1 = N - S0
ACC_R = 5016
TR0 = S0
TR1 = S1
K = 5760
KP = K + 16
C = 80
KCH = 2 * K // C


def _sc_partition(src_r, dst_r, w_r):
    mesh = plsc.VectorSubcoreMesh(core_axis_name="c", subcore_axis_name="s",
                                  num_cores=NC)

    @functools.partial(
        pl.kernel,
        mesh=mesh,
        out_type=[
            jax.ShapeDtypeStruct((NC * NW * K,), jnp.int32),
            jax.ShapeDtypeStruct((NC * NW * K,), jnp.int32),
            jax.ShapeDtypeStruct((NC * NW * K,), jnp.float32),
        ],
        scratch_types=[
            pltpu.VMEM((8, 128), jnp.int32),
            pltpu.VMEM((8, 128), jnp.int32),
            pltpu.VMEM((8, 128), jnp.float32),
            pltpu.VMEM((KP,), jnp.int32),
            pltpu.VMEM((KP,), jnp.int32),
            pltpu.VMEM((KP,), jnp.float32),
            pltpu.VMEM((KP,), jnp.int32),
            pltpu.VMEM((KP,), jnp.int32),
            pltpu.VMEM((KP,), jnp.float32),
        ],
    )
    def k(src_hbm, dst_hbm, w_hbm, osrc_hbm, odst_hbm, ow_hbm,
          src_c, dst_c, w_c, sl_b, dl_b, wl_b, sh_b, dh_b, wh_b):
        c = lax.axis_index("c")
        s = lax.axis_index("s")
        t = c * NS + s

        iota = lax.iota(jnp.int32, 16)
        one = jnp.ones((16,), jnp.int32)
        zero = jnp.zeros((16,), jnp.int32)
        zi = jnp.zeros((16,), jnp.int32)
        zf = jnp.zeros((16,), jnp.float32)
        trl = jnp.full((16,), TR0, jnp.int32)
        trh = jnp.full((16,), TR1, jnp.int32)

        def prefix_incl(v):
            cs = v
            for step in (1, 2, 4, 8):
                sh = _lane_gather(cs, jnp.maximum(iota - step, 0))
                cs = cs + jnp.where(iota >= step, sh, zero)
            return cs

        def lower_bound(cs):
            lo = jnp.full((16,), -1, jnp.int32)
            tgt = iota + 1
            for step in (8, 4, 2, 1):
                cand = lo + step
                v = _lane_gather(cs, jnp.clip(cand, 0, 15))
                ok = jnp.logical_and(cand <= 15, v < tgt)
                lo = jnp.where(ok, cand, lo)
            return jnp.clip(lo + 1, 0, 15)

        def stage_body(st, carry):
            pltpu.sync_copy(src_hbm.at[t, st], src_c)
            pltpu.sync_copy(dst_hbm.at[t, st], dst_c)
            pltpu.sync_copy(w_hbm.at[t, st], w_c)

            def group_body(g, offs):
                off_l, off_h = offs
                r = g // 8
                sl = pl.ds((g % 8) * 16, 16)
                s16 = src_c[r, sl]
                d16 = dst_c[r, sl]
                w16 = w_c[r, sl]
                ml = d16 < S0
                mli = jnp.where(ml, one, zero)
                cs = prefix_incl(mli)
                cnt = cs[15]
                sel_l = lower_bound(cs)
                sel_h = lower_bound((iota + 1) - cs)
                ol = jnp.minimum(off_l, KP - 16)
                oh = jnp.minimum(off_h, KP - 16)
                sl_b[pl.ds(ol, 16)] = _lane_gather(s16, sel_l)
                dl_b[pl.ds(ol, 16)] = _lane_gather(d16, sel_l)
                wl_b[pl.ds(ol, 16)] = _lane_gather(w16, sel_l)
                sh_b[pl.ds(oh, 16)] = _lane_gather(s16, sel_h)
                dh_b[pl.ds(oh, 16)] = _lane_gather(d16, sel_h) - S0
                wh_b[pl.ds(oh, 16)] = _lane_gather(w16, sel_h)
                return off_l + cnt, off_h + (16 - cnt)
            return lax.fori_loop(0, 64, group_body, carry)
        off_l, off_h = lax.fori_loop(0, NSTG, stage_body,
                                     (jnp.int32(0), jnp.int32(0)))

        def padl(g, o):
            @pl.when(o <= KP - 16)
            def _():
                oo = jnp.minimum(o, KP - 16)
                sl_b[pl.ds(oo, 16)] = zi
                dl_b[pl.ds(oo, 16)] = trl
                wl_b[pl.ds(oo, 16)] = zf
            return o + 16
        lax.fori_loop(0, KP // 16, padl, off_l)

        def padh(g, o):
            @pl.when(o <= KP - 16)
            def _():
                oo = jnp.minimum(o, KP - 16)
                sh_b[pl.ds(oo, 16)] = zi
                dh_b[pl.ds(oo, 16)] = trh
                wh_b[pl.ds(oo, 16)] = zf
            return o + 16
        lax.fori_loop(0, KP // 16, padh, off_h)

        lo = t * K
        hi = (NW + t) * K
        pltpu.sync_copy(sl_b.at[pl.ds(0, K)], osrc_hbm.at[pl.ds(lo, K)])
        pltpu.sync_copy(dl_b.at[pl.ds(0, K)], odst_hbm.at[pl.ds(lo, K)])
        pltpu.sync_copy(wl_b.at[pl.ds(0, K)], ow_hbm.at[pl.ds(lo, K)])
        pltpu.sync_copy(sh_b.at[pl.ds(0, K)], osrc_hbm.at[pl.ds(hi, K)])
        pltpu.sync_copy(dh_b.at[pl.ds(0, K)], odst_hbm.at[pl.ds(hi, K)])
        pltpu.sync_copy(wh_b.at[pl.ds(0, K)], ow_hbm.at[pl.ds(hi, K)])

    return k(src_r, dst_r, w_r)


def _sc_aggregate(hl, srcp, dstp, wp):
    mesh = plsc.VectorSubcoreMesh(core_axis_name="c", subcore_axis_name="s",
                                  num_cores=NC)

    @functools.partial(
        pl.kernel,
        mesh=mesh,
        out_type=jax.ShapeDtypeStruct((N, D), jnp.float32),
        scratch_types=[
            pltpu.VMEM((KCH, C), jnp.int32),
            pltpu.VMEM((KCH, C), jnp.int32),
            pltpu.VMEM((KCH, C), jnp.float32),
            pltpu.VMEM((2, C, D), jnp.float32),
            pltpu.VMEM_SHARED((ACC_R, D), jnp.float32),
            pltpu.SemaphoreType.DMA,
            pltpu.SemaphoreType.DMA,
        ],
    )
    def k(hl_hbm, src_hbm, dst_hbm, w_hbm, out_hbm,
          src_v, dst_v, w_v, rows_v, acc, gsem, ssem):
        c = lax.axis_index("c")
        s = lax.axis_index("s")

        def zrow_body(r, carry):
            for bb in range(2):
                for kk in range(D // 16):
                    rows_v[bb, r, pl.ds(kk * 16, 16)] = (
                        jnp.zeros((16,), jnp.float32))
            return carry
        lax.fori_loop(0, C, zrow_body, 0)
        for tt in range(4):
            pltpu.sync_copy(rows_v.at[0, pl.ds(0, 78)],
                            acc.at[pl.ds(s * 312 + tt * 78, 78)])

        @pl.when(s == 0)
        def _zero_rem():
            pltpu.sync_copy(rows_v.at[0, pl.ds(0, 24)],
                            acc.at[pl.ds(NS * 312, 24)])

        for r in range(2):
            blk = pl.ds(r * (K // C), K // C)
            pltpu.sync_copy(src_hbm.at[c, 2 * s + r], src_v.at[blk])
            pltpu.sync_copy(dst_hbm.at[c, 2 * s + r], dst_v.at[blk])
            pltpu.sync_copy(w_hbm.at[c, 2 * s + r], w_v.at[blk])
        plsc.subcore_barrier()

        def chunk_body(j, carry):
            pltpu.async_copy(hl_hbm.at[src_v.at[j]], rows_v.at[0],
                             gsem).wait()

            def group_body(g, cc):
                w16 = w_v[j, pl.ds(g * 16, 16)]
                for e in range(16):
                    ws = w16[e]
                    r = g * 16 + e
                    for kk in range(D // 16):
                        sl = pl.ds(kk * 16, 16)
                        rows_v[0, r, sl] = rows_v[0, r, sl] * ws
                return cc
            lax.fori_loop(0, C // 16, group_body, 0)

            pltpu.sync_copy(rows_v.at[0], acc.at[dst_v.at[j]], add=True)
            return carry
        lax.fori_loop(0, KCH, chunk_body, 0)

        plsc.subcore_barrier()
        pltpu.sync_copy(acc.at[pl.ds(s * 312, 312)],
                        out_hbm.at[pl.ds(c * S0 + s * 312, 312)])

        @pl.when(jnp.logical_and(c == 0, s == 0))
        def _write_rem():
            pltpu.sync_copy(acc.at[pl.ds(NS * 312, 16)],
                            out_hbm.at[pl.ds(NS * 312, 16)])

    return k(hl, srcp, dstp, wp)


_RB = 1000
_GRID = N // _RB


def _tc_first(x, w0t, b0, t0):
    def body(x_ref, w_ref, b_ref, t_ref, hl_ref, hid_ref):
        xv = x_ref[...]
        hid_ref[...] = t_ref[0, 0] * xv
        hl_ref[...] = (jnp.dot(xv, w_ref[...],
                               preferred_element_type=jnp.float32)
                       + b_ref[...])
    return pl.pallas_call(
        body,
        grid=(_GRID,),
        in_specs=[
            pl.BlockSpec((_RB, D), lambda i: (i, 0)),
            pl.BlockSpec((D, D), lambda i: (0, 0)),
            pl.BlockSpec((1, D), lambda i: (0, 0)),
            pl.BlockSpec((1, 1), lambda i: (0, 0)),
        ],
        out_specs=[
            pl.BlockSpec((_RB, D), lambda i: (i, 0)),
            pl.BlockSpec((_RB, D), lambda i: (i, 0)),
        ],
        out_shape=[
            jax.ShapeDtypeStruct((N, D), jnp.float32),
            jax.ShapeDtypeStruct((N, D), jnp.float32),
        ],
    )(x, w0t, b0, t0)


def _tc_mid(p, hidden, wt, bvec, t):
    def body(p_ref, hid_ref, w_ref, b_ref, t_ref, hl_ref, hido_ref):
        h = jnp.maximum(p_ref[...], 0.0)
        hido_ref[...] = hid_ref[...] + t_ref[0, 0] * h
        hl_ref[...] = (jnp.dot(h, w_ref[...],
                               preferred_element_type=jnp.float32)
                       + b_ref[...])
    return pl.pallas_call(
        body,
        grid=(_GRID,),
        in_specs=[
            pl.BlockSpec((_RB, D), lambda i: (i, 0)),
            pl.BlockSpec((_RB, D), lambda i: (i, 0)),
            pl.BlockSpec((D, D), lambda i: (0, 0)),
            pl.BlockSpec((1, D), lambda i: (0, 0)),
            pl.BlockSpec((1, 1), lambda i: (0, 0)),
        ],
        out_specs=[
            pl.BlockSpec((_RB, D), lambda i: (i, 0)),
            pl.BlockSpec((_RB, D), lambda i: (i, 0)),
        ],
        out_shape=[
            jax.ShapeDtypeStruct((N, D), jnp.float32),
            jax.ShapeDtypeStruct((N, D), jnp.float32),
        ],
    )(p, hidden, wt, bvec, t)


def _tc_last(p, hidden, t):
    def body(p_ref, hid_ref, t_ref, hido_ref):
        hido_ref[...] = hid_ref[...] + t_ref[0, 0] * jnp.maximum(
            p_ref[...], 0.0)
    return pl.pallas_call(
        body,
        grid=(_GRID,),
        in_specs=[
            pl.BlockSpec((_RB, D), lambda i: (i, 0)),
            pl.BlockSpec((_RB, D), lambda i: (i, 0)),
            pl.BlockSpec((1, 1), lambda i: (0, 0)),
        ],
        out_specs=pl.BlockSpec((_RB, D), lambda i: (i, 0)),
        out_shape=jax.ShapeDtypeStruct((N, D), jnp.float32),
    )(p, hidden, t)


def kernel(x, edge_index, edge_weight, W, b, temp):
    npad = NW * EPBP - E
    src_r = jnp.concatenate(
        [edge_index[0], jnp.zeros((npad,), jnp.int32)]).reshape(
            NW, NSTG, 8, 128)
    dst_r = jnp.concatenate(
        [edge_index[1], jnp.zeros((npad,), jnp.int32)]).reshape(
            NW, NSTG, 8, 128)
    w_r = jnp.concatenate(
        [edge_weight, jnp.zeros((npad,), jnp.float32)]).reshape(
            NW, NSTG, 8, 128)
    wt = jnp.swapaxes(W, 1, 2)
    b2 = b.reshape(L, 1, D)
    tc = temp.reshape(L + 1, 1, 1)

    srcp, dstp, wp = _sc_partition(src_r, dst_r, w_r)
    srcp = srcp.reshape(NC, NW, K // C, C)
    dstp = dstp.reshape(NC, NW, K // C, C)
    wp = wp.reshape(NC, NW, K // C, C)

    hl, hidden = _tc_first(x, wt[0], b2[0], tc[0])
    for i in range(1, L):
        p = _sc_aggregate(hl, srcp, dstp, wp)
        hl, hidden = _tc_mid(p, hidden, wt[i], b2[i], tc[i])
    p = _sc_aggregate(hl, srcp, dstp, wp)
    return _tc_last(p, hidden, tc[L])


# speedup vs baseline: 5.8349x; 5.8349x over previous
import functools

import jax
import jax.numpy as jnp
from jax import lax
from jax.experimental import pallas as pl
from jax.experimental.pallas import tpu as pltpu
from jax.experimental.pallas import tpu_sc as plsc

N = 10000
E = 320000
D = 128
L = 10

NS = 16
EPW = E // NS
C = 80
SB = 10
NSC = EPW // (SB * C)
RPT = 624
REM = N - NS * RPT


def _sc_aggregate(hl, src_r, dst_r, w_r):
    mesh = plsc.VectorSubcoreMesh(core_axis_name="c", subcore_axis_name="s",
                                  num_cores=1)

    @functools.partial(
        pl.kernel,
        mesh=mesh,
        out_type=jax.ShapeDtypeStruct((N, D), jnp.float32),
        scratch_types=[
            pltpu.VMEM((SB, C), jnp.int32),
            pltpu.VMEM((SB, C), jnp.int32),
            pltpu.VMEM((SB, C), jnp.float32),
            pltpu.VMEM((2, C, D), jnp.float32),
            pltpu.VMEM_SHARED((N, D), jnp.float32),
            pltpu.SemaphoreType.DMA,
        ],
    )
    def k(hl_hbm, src_hbm, dst_hbm, w_hbm, out_hbm,
          src_v, dst_v, w_v, rows_v, acc, gsem):
        s = lax.axis_index("s")

        def zrow_body(r, carry):
            for kk in range(D // 16):
                rows_v[0, r, pl.ds(kk * 16, 16)] = (
                    jnp.zeros((16,), jnp.float32))
            return carry
        lax.fori_loop(0, C, zrow_body, 0)
        for t in range(RPT // C):
            pltpu.sync_copy(rows_v.at[0],
                            acc.at[pl.ds(s * RPT + t * C, C)])
        pltpu.sync_copy(rows_v.at[0, pl.ds(0, RPT % C)],
                        acc.at[pl.ds(s * RPT + (RPT // C) * C, RPT % C)])

        @pl.when(s == 0)
        def _zero_rem():
            pltpu.sync_copy(rows_v.at[0, pl.ds(0, REM)],
                            acc.at[pl.ds(NS * RPT, REM)])
        plsc.subcore_barrier()

        def scale_chunk(b, j):
            def group_body(g, cc):
                w16 = w_v[j, pl.ds(g * 16, 16)]
                for e in range(16):
                    ws = w16[e]
                    r = g * 16 + e
                    for kk in range(D // 16):
                        sl = pl.ds(kk * 16, 16)
                        rows_v[b, r, sl] = rows_v[b, r, sl] * ws
                return cc
            lax.fori_loop(0, C // 16, group_body, 0)

        def super_body(t, carry):
            pltpu.sync_copy(src_hbm.at[s, t], src_v)
            pltpu.sync_copy(dst_hbm.at[s, t], dst_v)
            pltpu.sync_copy(w_hbm.at[s, t], w_v)

            h = [None] * SB
            h[0] = pltpu.async_copy(hl_hbm.at[src_v.at[0]], rows_v.at[0],
                                    gsem)
            for j in range(SB):
                b = j % 2
                if j + 1 < SB:
                    h[j + 1] = pltpu.async_copy(
                        hl_hbm.at[src_v.at[j + 1]], rows_v.at[1 - b], gsem)
                h[j].wait()
                scale_chunk(b, j)
                pltpu.sync_copy(rows_v.at[b], acc.at[dst_v.at[j]], add=True)
            return carry
        lax.fori_loop(0, NSC, super_body, 0)

        plsc.subcore_barrier()
        pltpu.sync_copy(acc.at[pl.ds(s * RPT, RPT)],
                        out_hbm.at[pl.ds(s * RPT, RPT)])

        @pl.when(s == 0)
        def _write_rem():
            pltpu.sync_copy(acc.at[pl.ds(NS * RPT, REM)],
                            out_hbm.at[pl.ds(NS * RPT, REM)])

    return k(hl, src_r, dst_r, w_r)


_RB = 1000
_GRID = N // _RB


def _tc_first(x, w0t, b0, t0):
    def body(x_ref, w_ref, b_ref, t_ref, hl_ref, hid_ref):
        xv = x_ref[...]
        hid_ref[...] = t_ref[0, 0] * xv
        hl_ref[...] = (jnp.dot(xv, w_ref[...],
                               preferred_element_type=jnp.float32)
                       + b_ref[...])
    return pl.pallas_call(
        body,
        grid=(_GRID,),
        in_specs=[
            pl.BlockSpec((_RB, D), lambda i: (i, 0)),
            pl.BlockSpec((D, D), lambda i: (0, 0)),
            pl.BlockSpec((1, D), lambda i: (0, 0)),
            pl.BlockSpec((1, 1), lambda i: (0, 0)),
        ],
        out_specs=[
            pl.BlockSpec((_RB, D), lambda i: (i, 0)),
            pl.BlockSpec((_RB, D), lambda i: (i, 0)),
        ],
        out_shape=[
            jax.ShapeDtypeStruct((N, D), jnp.float32),
            jax.ShapeDtypeStruct((N, D), jnp.float32),
        ],
    )(x, w0t, b0, t0)


def _tc_mid(p, hidden, wt, bvec, t):
    def body(p_ref, hid_ref, w_ref, b_ref, t_ref, hl_ref, hido_ref):
        h = jnp.maximum(p_ref[...], 0.0)
        hido_ref[...] = hid_ref[...] + t_ref[0, 0] * h
        hl_ref[...] = (jnp.dot(h, w_ref[...],
                               preferred_element_type=jnp.float32)
                       + b_ref[...])
    return pl.pallas_call(
        body,
        grid=(_GRID,),
        in_specs=[
            pl.BlockSpec((_RB, D), lambda i: (i, 0)),
            pl.BlockSpec((_RB, D), lambda i: (i, 0)),
            pl.BlockSpec((D, D), lambda i: (0, 0)),
            pl.BlockSpec((1, D), lambda i: (0, 0)),
            pl.BlockSpec((1, 1), lambda i: (0, 0)),
        ],
        out_specs=[
            pl.BlockSpec((_RB, D), lambda i: (i, 0)),
            pl.BlockSpec((_RB, D), lambda i: (i, 0)),
        ],
        out_shape=[
            jax.ShapeDtypeStruct((N, D), jnp.float32),
            jax.ShapeDtypeStruct((N, D), jnp.float32),
        ],
    )(p, hidden, wt, bvec, t)


def _tc_last(p, hidden, t):
    def body(p_ref, hid_ref, t_ref, hido_ref):
        hido_ref[...] = hid_ref[...] + t_ref[0, 0] * jnp.maximum(
            p_ref[...], 0.0)
    return pl.pallas_call(
        body,
        grid=(_GRID,),
        in_specs=[
            pl.BlockSpec((_RB, D), lambda i: (i, 0)),
            pl.BlockSpec((_RB, D), lambda i: (i, 0)),
            pl.BlockSpec((1, 1), lambda i: (0, 0)),
        ],
        out_specs=pl.BlockSpec((_RB, D), lambda i: (i, 0)),
        out_shape=jax.ShapeDtypeStruct((N, D), jnp.float32),
    )(p, hidden, t)


def kernel(x, edge_index, edge_weight, W, b, temp):
    src_r = edge_index[0].reshape(NS, NSC, SB, C)
    dst_r = edge_index[1].reshape(NS, NSC, SB, C)
    w_r = edge_weight.reshape(NS, NSC, SB, C)
    wt = jnp.swapaxes(W, 1, 2)
    b2 = b.reshape(L, 1, D)
    tc = temp.reshape(L + 1, 1, 1)

    hl, hidden = _tc_first(x, wt[0], b2[0], tc[0])
    for i in range(1, L):
        p = _sc_aggregate(hl, src_r, dst_r, w_r)
        hl, hidden = _tc_mid(p, hidden, wt[i], b2[i], tc[i])
    p = _sc_aggregate(hl, src_r, dst_r, w_r)
    return _tc_last(p, hidden, tc[L])


# speedup vs baseline: 6.3240x; 1.0838x over previous
import functools

import jax
import jax.numpy as jnp
from jax import lax
from jax.experimental import pallas as pl
from jax.experimental.pallas import tpu as pltpu
from jax.experimental.pallas import tpu_sc as plsc

N = 10000
E = 320000
D = 128
L = 10

NS = 16
EPW = E // NS
C = 80
SB = 10
NSC = EPW // (SB * C)
RPT = 624
REM = N - NS * RPT


def _sc_aggregate(hl, src_r, dst_r, w_r):
    mesh = plsc.VectorSubcoreMesh(core_axis_name="c", subcore_axis_name="s",
                                  num_cores=1)

    @functools.partial(
        pl.kernel,
        mesh=mesh,
        out_type=jax.ShapeDtypeStruct((N, D), jnp.float32),
        scratch_types=[
            pltpu.VMEM((SB, C), jnp.int32),
            pltpu.VMEM((SB, C), jnp.int32),
            pltpu.VMEM((SB, C), jnp.float32),
            pltpu.VMEM((3, C, D), jnp.float32),
            pltpu.VMEM_SHARED((N, D), jnp.float32),
            pltpu.SemaphoreType.DMA,
            pltpu.SemaphoreType.DMA,
        ],
    )
    def k(hl_hbm, src_hbm, dst_hbm, w_hbm, out_hbm,
          src_v, dst_v, w_v, rows_v, acc, gsem, ssem):
        s = lax.axis_index("s")

        def zrow_body(r, carry):
            for kk in range(D // 16):
                rows_v[0, r, pl.ds(kk * 16, 16)] = (
                    jnp.zeros((16,), jnp.float32))
            return carry
        lax.fori_loop(0, C, zrow_body, 0)
        for t in range(RPT // C):
            pltpu.sync_copy(rows_v.at[0],
                            acc.at[pl.ds(s * RPT + t * C, C)])
        pltpu.sync_copy(rows_v.at[0, pl.ds(0, RPT % C)],
                        acc.at[pl.ds(s * RPT + (RPT // C) * C, RPT % C)])

        @pl.when(s == 0)
        def _zero_rem():
            pltpu.sync_copy(rows_v.at[0, pl.ds(0, REM)],
                            acc.at[pl.ds(NS * RPT, REM)])
        plsc.subcore_barrier()

        def scale_chunk(b, j):
            def group_body(g, cc):
                w16 = w_v[j, pl.ds(g * 16, 16)]
                for e in range(16):
                    ws = w16[e]
                    r = g * 16 + e
                    for kk in range(D // 16):
                        sl = pl.ds(kk * 16, 16)
                        rows_v[b, r, sl] = rows_v[b, r, sl] * ws
                return cc
            lax.fori_loop(0, C // 16, group_body, 0)

        def super_body(t, carry):
            pltpu.sync_copy(src_hbm.at[s, t], src_v)
            pltpu.sync_copy(dst_hbm.at[s, t], dst_v)
            pltpu.sync_copy(w_hbm.at[s, t], w_v)

            h = [None] * SB
            sc = [None] * SB
            h[0] = pltpu.async_copy(hl_hbm.at[src_v.at[0]], rows_v.at[0],
                                    gsem)
            h[1] = pltpu.async_copy(hl_hbm.at[src_v.at[1]], rows_v.at[1],
                                    gsem)
            for j in range(SB):
                b = j % 3
                if j + 2 < SB:
                    if j >= 1:
                        sc[j - 1].wait()
                    h[j + 2] = pltpu.async_copy(
                        hl_hbm.at[src_v.at[j + 2]], rows_v.at[(j + 2) % 3],
                        gsem)
                h[j].wait()
                scale_chunk(b, j)
                sc[j] = pltpu.async_copy(rows_v.at[b], acc.at[dst_v.at[j]],
                                         ssem, add=True)
            sc[SB - 3].wait()
            sc[SB - 2].wait()
            sc[SB - 1].wait()
            return carry
        lax.fori_loop(0, NSC, super_body, 0)

        plsc.subcore_barrier()
        pltpu.sync_copy(acc.at[pl.ds(s * RPT, RPT)],
                        out_hbm.at[pl.ds(s * RPT, RPT)])

        @pl.when(s == 0)
        def _write_rem():
            pltpu.sync_copy(acc.at[pl.ds(NS * RPT, REM)],
                            out_hbm.at[pl.ds(NS * RPT, REM)])

    return k(hl, src_r, dst_r, w_r)


_RB = 1000
_GRID = N // _RB


def _tc_first(x, w0t, b0, t0):
    def body(x_ref, w_ref, b_ref, t_ref, hl_ref, hid_ref):
        xv = x_ref[...]
        hid_ref[...] = t_ref[0, 0] * xv
        hl_ref[...] = (jnp.dot(xv, w_ref[...],
                               preferred_element_type=jnp.float32)
                       + b_ref[...])
    return pl.pallas_call(
        body,
        grid=(_GRID,),
        in_specs=[
            pl.BlockSpec((_RB, D), lambda i: (i, 0)),
            pl.BlockSpec((D, D), lambda i: (0, 0)),
            pl.BlockSpec((1, D), lambda i: (0, 0)),
            pl.BlockSpec((1, 1), lambda i: (0, 0)),
        ],
        out_specs=[
            pl.BlockSpec((_RB, D), lambda i: (i, 0)),
            pl.BlockSpec((_RB, D), lambda i: (i, 0)),
        ],
        out_shape=[
            jax.ShapeDtypeStruct((N, D), jnp.float32),
            jax.ShapeDtypeStruct((N, D), jnp.float32),
        ],
    )(x, w0t, b0, t0)


def _tc_mid(p, hidden, wt, bvec, t):
    def body(p_ref, hid_ref, w_ref, b_ref, t_ref, hl_ref, hido_ref):
        h = jnp.maximum(p_ref[...], 0.0)
        hido_ref[...] = hid_ref[...] + t_ref[0, 0] * h
        hl_ref[...] = (jnp.dot(h, w_ref[...],
                               preferred_element_type=jnp.float32)
                       + b_ref[...])
    return pl.pallas_call(
        body,
        grid=(_GRID,),
        in_specs=[
            pl.BlockSpec((_RB, D), lambda i: (i, 0)),
            pl.BlockSpec((_RB, D), lambda i: (i, 0)),
            pl.BlockSpec((D, D), lambda i: (0, 0)),
            pl.BlockSpec((1, D), lambda i: (0, 0)),
            pl.BlockSpec((1, 1), lambda i: (0, 0)),
        ],
        out_specs=[
            pl.BlockSpec((_RB, D), lambda i: (i, 0)),
            pl.BlockSpec((_RB, D), lambda i: (i, 0)),
        ],
        out_shape=[
            jax.ShapeDtypeStruct((N, D), jnp.float32),
            jax.ShapeDtypeStruct((N, D), jnp.float32),
        ],
    )(p, hidden, wt, bvec, t)


def _tc_last(p, hidden, t):
    def body(p_ref, hid_ref, t_ref, hido_ref):
        hido_ref[...] = hid_ref[...] + t_ref[0, 0] * jnp.maximum(
            p_ref[...], 0.0)
    return pl.pallas_call(
        body,
        grid=(_GRID,),
        in_specs=[
            pl.BlockSpec((_RB, D), lambda i: (i, 0)),
            pl.BlockSpec((_RB, D), lambda i: (i, 0)),
            pl.BlockSpec((1, 1), lambda i: (0, 0)),
        ],
        out_specs=pl.BlockSpec((_RB, D), lambda i: (i, 0)),
        out_shape=jax.ShapeDtypeStruct((N, D), jnp.float32),
    )(p, hidden, t)


def kernel(x, edge_index, edge_weight, W, b, temp):
    src_r = edge_index[0].reshape(NS, NSC, SB, C)
    dst_r = edge_index[1].reshape(NS, NSC, SB, C)
    w_r = edge_weight.reshape(NS, NSC, SB, C)
    wt = jnp.swapaxes(W, 1, 2)
    b2 = b.reshape(L, 1, D)
    tc = temp.reshape(L + 1, 1, 1)

    hl, hidden = _tc_first(x, wt[0], b2[0], tc[0])
    for i in range(1, L):
        p = _sc_aggregate(hl, src_r, dst_r, w_r)
        hl, hidden = _tc_mid(p, hidden, wt[i], b2[i], tc[i])
    p = _sc_aggregate(hl, src_r, dst_r, w_r)
    return _tc_last(p, hidden, tc[L])
